# Initial kernel scaffold; baseline (speedup 1.0000x reference)
import jax
import jax.numpy as jnp
from jax.experimental import pallas as pl


def kernel(input_ids, context, token_embed, label_embed, W, b):
    raise NotImplementedError("write your pallas kernel here")


# speedup vs baseline: 3.2437x; 3.2437x over previous
import jax
import jax.numpy as jnp
from jax.experimental import pallas as pl
from jax.experimental.pallas import tpu as pltpu
from jax.experimental.pallas import tpu_sc as plsc

_B, _L = 1024, 32
_V, _D, _LV = 8192, 64, 1000
_BL = _B * _L

_DP = 128
_TOK_WIN = 128
_CTX_WIN = 128

_ROWS = 256
_NBATCH = _ROWS // _L


def _sc_gather_body(tok_hbm, ids_hbm, lab_hbm, ctx_hbm, otok_hbm, olab_hbm):
    def tok_body(i_vmem, o_vmem):
        pltpu.sync_copy(tok_hbm.at[i_vmem.at[0]], o_vmem)

    pltpu.emit_pipeline(
        tok_body,
        grid=(_BL // _TOK_WIN,),
        in_specs=[pl.BlockSpec((1, _TOK_WIN), index_map=lambda i: (0, i))],
        out_specs=[pl.BlockSpec((_TOK_WIN, _DP), index_map=lambda i: (i, 0))],
        core_axis_name=("c", "s"),
        dimension_semantics=(pltpu.PARALLEL,),
    )(ids_hbm, otok_hbm)

    def lab_body(i_vmem, o_vmem):
        pltpu.sync_copy(lab_hbm.at[i_vmem.at[0]], o_vmem)

    pltpu.emit_pipeline(
        lab_body,
        grid=(_B // _CTX_WIN,),
        in_specs=[pl.BlockSpec((1, _CTX_WIN), index_map=lambda i: (0, i))],
        out_specs=[pl.BlockSpec((_CTX_WIN, _DP), index_map=lambda i: (i, 0))],
        core_axis_name=("c", "s"),
        dimension_semantics=(pltpu.PARALLEL,),
    )(ctx_hbm, olab_hbm)


def _sc_gather(tok_bf, ids_flat, lab_bf, ctx_flat):
    f = pl.kernel(
        _sc_gather_body,
        out_type=(
            jax.ShapeDtypeStruct((_BL, _DP), jnp.float32),
            jax.ShapeDtypeStruct((_B, _DP), jnp.float32),
        ),
        mesh=plsc.VectorSubcoreMesh(core_axis_name="c", subcore_axis_name="s"),
    )
    return f(tok_bf, ids_flat, lab_bf, ctx_flat)


def _proj_body(tok_ref, lab_ref, wt_ref, b_ref, o_ref):
    tok = tok_ref[...][:, :_D].reshape(_NBATCH, _L, _D)
    lab = lab_ref[...][:, :_D]
    x = (tok + lab[:, None, :]).reshape(_ROWS, _D).astype(jnp.bfloat16)
    acc = jnp.dot(x, wt_ref[...], preferred_element_type=jnp.float32)
    o_ref[...] = acc + b_ref[...]


def _project(tok_x, lab_x, wt, b2d):
    return pl.pallas_call(
        _proj_body,
        grid=(_BL // _ROWS,),
        in_specs=[
            pl.BlockSpec((_ROWS, _DP), lambda i: (i, 0)),
            pl.BlockSpec((_NBATCH, _DP), lambda i: (i, 0)),
            pl.BlockSpec((_D, _V), lambda i: (0, 0)),
            pl.BlockSpec((1, _V), lambda i: (0, 0)),
        ],
        out_specs=pl.BlockSpec((_ROWS, _V), lambda i: (i, 0)),
        out_shape=jax.ShapeDtypeStruct((_BL, _V), jnp.float32),
        compiler_params=pltpu.CompilerParams(
            dimension_semantics=("arbitrary",),
        ),
    )(tok_x, lab_x, wt, b2d)


def kernel(input_ids, context, token_embed, label_embed, W, b):
    wt = W.astype(jnp.bfloat16).T
    ids_flat = input_ids.reshape(1, _BL).astype(jnp.int32)
    ctx_flat = context.reshape(1, _B).astype(jnp.int32)
    tok_pad = jnp.pad(token_embed, ((0, 0), (0, _DP - _D)))
    lab_pad = jnp.pad(label_embed, ((0, 0), (0, _DP - _D)))
    tok_x, lab_x = _sc_gather(tok_pad, ids_flat, lab_pad, ctx_flat)
    logits = _project(tok_x, lab_x, wt, b.reshape(1, _V))
    return logits.reshape(_B, _L, _V)


# speedup vs baseline: 3.2652x; 1.0066x over previous
import jax
import jax.numpy as jnp
from jax.experimental import pallas as pl
from jax.experimental.pallas import tpu as pltpu
from jax.experimental.pallas import tpu_sc as plsc

_B, _L = 1024, 32
_V, _D, _LV = 8192, 64, 1000
_BL = _B * _L

_DP = 128
_TOK_WIN = 128
_CTX_WIN = 128

_ROWS = 512
_NBATCH = _ROWS // _L


def _sc_gather_body(tok_hbm, ids_hbm, lab_hbm, ctx_hbm, otok_hbm, olab_hbm):
    def tok_body(i_vmem, o_vmem):
        pltpu.sync_copy(tok_hbm.at[i_vmem.at[0]], o_vmem)

    pltpu.emit_pipeline(
        tok_body,
        grid=(_BL // _TOK_WIN,),
        in_specs=[pl.BlockSpec((1, _TOK_WIN), index_map=lambda i: (0, i))],
        out_specs=[pl.BlockSpec((_TOK_WIN, _DP), index_map=lambda i: (i, 0))],
        core_axis_name=("c", "s"),
        dimension_semantics=(pltpu.PARALLEL,),
    )(ids_hbm, otok_hbm)

    def lab_body(i_vmem, o_vmem):
        pltpu.sync_copy(lab_hbm.at[i_vmem.at[0]], o_vmem)

    pltpu.emit_pipeline(
        lab_body,
        grid=(_B // _CTX_WIN,),
        in_specs=[pl.BlockSpec((1, _CTX_WIN), index_map=lambda i: (0, i))],
        out_specs=[pl.BlockSpec((_CTX_WIN, _DP), index_map=lambda i: (i, 0))],
        core_axis_name=("c", "s"),
        dimension_semantics=(pltpu.PARALLEL,),
    )(ctx_hbm, olab_hbm)


def _sc_gather(tok_bf, ids_flat, lab_bf, ctx_flat):
    f = pl.kernel(
        _sc_gather_body,
        out_type=(
            jax.ShapeDtypeStruct((_BL, _DP), jnp.float32),
            jax.ShapeDtypeStruct((_B, _DP), jnp.float32),
        ),
        mesh=plsc.VectorSubcoreMesh(core_axis_name="c", subcore_axis_name="s"),
    )
    return f(tok_bf, ids_flat, lab_bf, ctx_flat)


def _proj_body(tok_ref, lab_ref, wt_ref, b_ref, o_ref):
    tok = tok_ref[...][:, :_D].reshape(_NBATCH, _L, _D)
    lab = lab_ref[...][:, :_D]
    x = (tok + lab[:, None, :]).reshape(_ROWS, _D).astype(jnp.bfloat16)
    acc = jnp.dot(x, wt_ref[...], preferred_element_type=jnp.float32)
    o_ref[...] = acc + b_ref[...]


def _project(tok_x, lab_x, wt, b2d):
    return pl.pallas_call(
        _proj_body,
        grid=(_BL // _ROWS,),
        in_specs=[
            pl.BlockSpec((_ROWS, _DP), lambda i: (i, 0)),
            pl.BlockSpec((_NBATCH, _DP), lambda i: (i, 0)),
            pl.BlockSpec((_D, _V), lambda i: (0, 0)),
            pl.BlockSpec((1, _V), lambda i: (0, 0)),
        ],
        out_specs=pl.BlockSpec((_ROWS, _V), lambda i: (i, 0)),
        out_shape=jax.ShapeDtypeStruct((_BL, _V), jnp.float32),
        compiler_params=pltpu.CompilerParams(
            dimension_semantics=("arbitrary",),
        ),
    )(tok_x, lab_x, wt, b2d)


def kernel(input_ids, context, token_embed, label_embed, W, b):
    wt = W.astype(jnp.bfloat16).T
    ids_flat = input_ids.reshape(1, _BL).astype(jnp.int32)
    ctx_flat = context.reshape(1, _B).astype(jnp.int32)
    tok_pad = jnp.pad(token_embed, ((0, 0), (0, _DP - _D)))
    lab_pad = jnp.pad(label_embed, ((0, 0), (0, _DP - _D)))
    tok_x, lab_x = _sc_gather(tok_pad, ids_flat, lab_pad, ctx_flat)
    logits = _project(tok_x, lab_x, wt, b.reshape(1, _V))
    return logits.reshape(_B, _L, _V)
